# baseline (speedup 1.0000x reference)
import functools
import math

import jax
import jax.numpy as jnp
from jax import lax
from jax.experimental import pallas as pl
from jax.experimental.pallas import tpu as pltpu
from jax.experimental.pallas import tpu_sc as plsc

_INTERVAL = 0.001
_NUM_COL = int(math.ceil(1.0 / _INTERVAL))
_EMBED = 32
_B = 16384
_P = 50
_TOTAL = _B * _P

_NC = 2
_NS = 16
_NW = _NC * _NS

_NB = _B // 128
_TCOLS = _P * _NB
_PER_W = _TCOLS // _NW

_GDEPTH = 16
_ODEPTH = 8

_mesh = plsc.VectorSubcoreMesh(core_axis_name="c", subcore_axis_name="s")


def _idx_body(x_ref, y_ref, o_ref):
    col = jnp.clip(jnp.floor(x_ref[...] / _INTERVAL), 0, _NUM_COL - 1)
    row = jnp.clip(jnp.floor(y_ref[...] / _INTERVAL), 0, _NUM_COL - 1)
    o_ref[...] = row.astype(jnp.int32) * _NUM_COL + col.astype(jnp.int32)


_idx_tc = pl.pallas_call(
    _idx_body,
    grid=(8,),
    in_specs=[
        pl.BlockSpec((_TCOLS // 8, 128), lambda i: (i, 0)),
        pl.BlockSpec((_TCOLS // 8, 128), lambda i: (i, 0)),
    ],
    out_specs=pl.BlockSpec((_TCOLS // 8, 128), lambda i: (i, 0)),
    out_shape=jax.ShapeDtypeStruct((_TCOLS, 128), jnp.int32),
)


@functools.partial(
    pl.kernel,
    mesh=_mesh,
    out_type=jax.ShapeDtypeStruct((_TOTAL * _EMBED,), jnp.float32),
    scratch_types=[
        pltpu.VMEM((_PER_W, 128), jnp.int32),
        pltpu.VMEM((_GDEPTH * 128, _EMBED), jnp.float32),
        pltpu.VMEM((_ODEPTH * 4096,), jnp.float32),
        pltpu.SemaphoreType.DMA,
        pltpu.SemaphoreType.DMA,
        pltpu.SemaphoreType.DMA,
    ],
    compiler_params=pltpu.CompilerParams(
        use_tc_tiling_on_sc=False, needs_layout_passes=False
    ),
)
def _lookup(idx_hbm, table_hbm, out_hbm, idxv, rows, rowsT,
            sem_in, sem_g, sem_out):
    wid = lax.axis_index("s") * _NC + lax.axis_index("c")
    t0 = wid * _PER_W
    lanes = lax.iota(jnp.int32, 16)
    nvecs = [n0 + lanes for n0 in range(0, 128, 16)]
    zeros = jnp.zeros((16,), jnp.int32)

    pltpu.async_copy(idx_hbm.at[pl.ds(t0, _PER_W), :], idxv, sem_in).wait()

    def fire_gather(t, slot):
        return pltpu.async_copy(
            table_hbm.at[idxv.at[t]],
            rows.at[pl.ds(slot * 128, 128), :], sem_g)

    def transpose(slot, oslot):
        rvecs = [slot * 128 + nv for nv in nvecs]
        ovecs = [oslot * 4096 + nv for nv in nvecs]

        def tr_body(c, carry):
            dv = (lanes + c) & 31
            pv = dv * 128
            for i in range(8):
                v = plsc.load_gather(rows, [rvecs[i], dv])
                plsc.store_scatter(rowsT, [pv + ovecs[i]], v)
            return carry

        lax.fori_loop(0, _EMBED, tr_body, 0)

    def fire_outs(t, oslot):
        copies = []
        p = t >> 7
        nb = t & 127
        r0 = p * (4 * _NB) + nb
        for db in range(4):
            copies.append(pltpu.async_copy(
                rowsT.at[pl.ds(oslot * 4096 + db * 1024, 1024)],
                out_hbm.at[pl.ds((r0 + db * _NB) * 1024, 1024)],
                sem_out))
        return copies

    prime = [fire_gather(t, t) for t in range(_GDEPTH)]
    for t in range(_ODEPTH):
        prime[t].wait()
        transpose(t, t)
        fire_outs(t0 + t, t)
        fire_gather(t + _GDEPTH, t)

    def steady(t, carry):
        slot = t & (_GDEPTH - 1)
        oslot = t & (_ODEPTH - 1)
        pltpu.make_async_copy(
            table_hbm.at[idxv.at[t]],
            rows.at[pl.ds(slot * 128, 128), :], sem_g).wait()
        pltpu.make_async_copy(
            rowsT.at[pl.ds(0, 4096)], out_hbm.at[pl.ds(0, 4096)], sem_out).wait()
        transpose(slot, oslot)
        fire_outs(t0 + t, oslot)
        fire_gather(t + _GDEPTH, slot)
        return carry

    lax.fori_loop(_ODEPTH, _PER_W - _GDEPTH, steady, 0)

    for t in range(_PER_W - _GDEPTH, _PER_W):
        slot = t % _GDEPTH
        oslot = t % _ODEPTH
        pltpu.make_async_copy(
            table_hbm.at[idxv.at[t]],
            rows.at[pl.ds(slot * 128, 128), :], sem_g).wait()
        pltpu.make_async_copy(
            rowsT.at[pl.ds(0, 4096)], out_hbm.at[pl.ds(0, 4096)], sem_out).wait()
        transpose(slot, oslot)
        fire_outs(t0 + t, oslot)

    for _ in range(_ODEPTH):
        pltpu.make_async_copy(
            rowsT.at[pl.ds(0, 4096)], out_hbm.at[pl.ds(0, 4096)], sem_out).wait()


def kernel(coords, W):
    x = coords[:, :, 0].T.reshape(_TCOLS, 128)
    y = coords[:, :, 1].T.reshape(_TCOLS, 128)
    idx = _idx_tc(x, y)
    out = _lookup(idx, W)
    out = out.reshape(_P, 4, _NB, 8, 128)
    out = out.transpose(2, 4, 0, 1, 3).reshape(_B, _P, _EMBED)
    return out


# speedup vs baseline: 1.0454x; 1.0454x over previous
import functools
import math

import jax
import jax.numpy as jnp
from jax import lax
from jax.experimental import pallas as pl
from jax.experimental.pallas import tpu as pltpu
from jax.experimental.pallas import tpu_sc as plsc

_INTERVAL = 0.001
_NUM_COL = int(math.ceil(1.0 / _INTERVAL))
_EMBED = 32
_B = 16384
_P = 50
_TOTAL = _B * _P

_NC = 2
_NS = 16
_NW = _NC * _NS

_NB = _B // 128
_TCOLS = _P * _NB
_PER_W = _TCOLS // _NW

_GDEPTH = 16
_ODEPTH = 8

_mesh = plsc.VectorSubcoreMesh(core_axis_name="c", subcore_axis_name="s")


def _idx_body(x_ref, y_ref, o_ref):
    col = jnp.clip(jnp.floor(x_ref[...] / _INTERVAL), 0, _NUM_COL - 1)
    row = jnp.clip(jnp.floor(y_ref[...] / _INTERVAL), 0, _NUM_COL - 1)
    o_ref[...] = row.astype(jnp.int32) * _NUM_COL + col.astype(jnp.int32)


_idx_tc = pl.pallas_call(
    _idx_body,
    grid=(8,),
    in_specs=[
        pl.BlockSpec((_TCOLS // 8, 128), lambda i: (i, 0)),
        pl.BlockSpec((_TCOLS // 8, 128), lambda i: (i, 0)),
    ],
    out_specs=pl.BlockSpec((_TCOLS // 8, 128), lambda i: (i, 0)),
    out_shape=jax.ShapeDtypeStruct((_TCOLS, 128), jnp.int32),
)

_NROWS = 1000 * 1000
_WBLK = 8192


def _wt_body(wt_ref, o_ref):
    o_ref[...] = wt_ref[...].T


_wtrans = pl.pallas_call(
    _wt_body,
    grid=(pl.cdiv(_NROWS, _WBLK),),
    in_specs=[pl.BlockSpec((_EMBED, _WBLK), lambda i: (0, i))],
    out_specs=pl.BlockSpec((_WBLK, _EMBED), lambda i: (i, 0)),
    out_shape=jax.ShapeDtypeStruct((_NROWS, _EMBED), jnp.float32),
)


@functools.partial(
    pl.kernel,
    mesh=_mesh,
    out_type=jax.ShapeDtypeStruct((_TOTAL * _EMBED,), jnp.float32),
    scratch_types=[
        pltpu.VMEM((_PER_W, 128), jnp.int32),
        pltpu.VMEM((_GDEPTH * 128, _EMBED), jnp.float32),
        pltpu.VMEM((_ODEPTH * 4096,), jnp.float32),
        pltpu.SemaphoreType.DMA,
        pltpu.SemaphoreType.DMA,
        pltpu.SemaphoreType.DMA,
    ],
    compiler_params=pltpu.CompilerParams(
        use_tc_tiling_on_sc=False, needs_layout_passes=False
    ),
)
def _lookup(idx_hbm, table_hbm, out_hbm, idxv, rows, rowsT,
            sem_in, sem_g, sem_out):
    wid = lax.axis_index("s") * _NC + lax.axis_index("c")
    t0 = wid * _PER_W
    lanes = lax.iota(jnp.int32, 16)
    nvecs = [n0 + lanes for n0 in range(0, 128, 16)]
    zeros = jnp.zeros((16,), jnp.int32)

    pltpu.async_copy(idx_hbm.at[pl.ds(t0, _PER_W), :], idxv, sem_in).wait()

    def fire_gather(t, slot):
        return pltpu.async_copy(
            table_hbm.at[idxv.at[t]],
            rows.at[pl.ds(slot * 128, 128), :], sem_g)

    def transpose(slot, oslot):
        rvecs = [slot * 128 + nv for nv in nvecs]
        ovecs = [oslot * 4096 + nv for nv in nvecs]

        def tr_body(c, carry):
            dv = (lanes + c) & 31
            pv = dv * 128
            vs = [plsc.load_gather(rows, [rvecs[i], dv]) for i in range(8)]
            for i in range(8):
                plsc.store_scatter(rowsT, [pv + ovecs[i]], vs[i])
            return carry

        lax.fori_loop(0, _EMBED, tr_body, 0, unroll=2)

    def fire_outs(t, oslot):
        copies = []
        p = t >> 7
        nb = t & 127
        r0 = p * (4 * _NB) + nb
        for db in range(4):
            copies.append(pltpu.async_copy(
                rowsT.at[pl.ds(oslot * 4096 + db * 1024, 1024)],
                out_hbm.at[pl.ds((r0 + db * _NB) * 1024, 1024)],
                sem_out))
        return copies

    prime = [fire_gather(t, t) for t in range(_GDEPTH)]
    for t in range(_ODEPTH):
        prime[t].wait()
        transpose(t, t)
        fire_outs(t0 + t, t)
        fire_gather(t + _GDEPTH, t)

    def steady(t, carry):
        slot = t & (_GDEPTH - 1)
        oslot = t & (_ODEPTH - 1)
        pltpu.make_async_copy(
            table_hbm.at[idxv.at[t]],
            rows.at[pl.ds(slot * 128, 128), :], sem_g).wait()
        pltpu.make_async_copy(
            rowsT.at[pl.ds(0, 4096)], out_hbm.at[pl.ds(0, 4096)], sem_out).wait()
        transpose(slot, oslot)
        fire_outs(t0 + t, oslot)
        fire_gather(t + _GDEPTH, slot)
        return carry

    lax.fori_loop(_ODEPTH, _PER_W - _GDEPTH, steady, 0)

    for t in range(_PER_W - _GDEPTH, _PER_W):
        slot = t % _GDEPTH
        oslot = t % _ODEPTH
        pltpu.make_async_copy(
            table_hbm.at[idxv.at[t]],
            rows.at[pl.ds(slot * 128, 128), :], sem_g).wait()
        pltpu.make_async_copy(
            rowsT.at[pl.ds(0, 4096)], out_hbm.at[pl.ds(0, 4096)], sem_out).wait()
        transpose(slot, oslot)
        fire_outs(t0 + t, oslot)

    for _ in range(_ODEPTH):
        pltpu.make_async_copy(
            rowsT.at[pl.ds(0, 4096)], out_hbm.at[pl.ds(0, 4096)], sem_out).wait()


def kernel(coords, W):
    x = coords[:, :, 0].T.reshape(_TCOLS, 128)
    y = coords[:, :, 1].T.reshape(_TCOLS, 128)
    idx = _idx_tc(x, y)
    w_rm = _wtrans(W.T)
    out = _lookup(idx, w_rm)
    out = out.reshape(_P, 4, _NB, 8, 128)
    out = out.transpose(2, 4, 0, 1, 3).reshape(_B, _P, _EMBED)
    return out


# speedup vs baseline: 1.0793x; 1.0325x over previous
import functools
import math

import jax
import jax.numpy as jnp
from jax import lax
from jax.experimental import pallas as pl
from jax.experimental.pallas import tpu as pltpu
from jax.experimental.pallas import tpu_sc as plsc

_INTERVAL = 0.001
_NUM_COL = int(math.ceil(1.0 / _INTERVAL))
_EMBED = 32
_B = 16384
_P = 50
_TOTAL = _B * _P

_NC = 2
_NS = 16
_NW = _NC * _NS

_NB = _B // 128
_TCOLS = _P * _NB
_PER_W = _TCOLS // _NW

_GDEPTH = 16
_ODEPTH = 8

_mesh = plsc.VectorSubcoreMesh(core_axis_name="c", subcore_axis_name="s")


def _idx_body(c_ref, o_ref):
    t = jnp.floor(c_ref[...] / _INTERVAL)
    t = jnp.clip(t, 0, _NUM_COL - 1).astype(jnp.int32)
    o_ref[...] = t[:, 1, :] * _NUM_COL + t[:, 0, :]


_idx_tc = pl.pallas_call(
    _idx_body,
    grid=(8,),
    in_specs=[
        pl.BlockSpec((_TCOLS // 8, 2, 128), lambda i: (i, 0, 0)),
    ],
    out_specs=pl.BlockSpec((_TCOLS // 8, 128), lambda i: (i, 0)),
    out_shape=jax.ShapeDtypeStruct((_TCOLS, 128), jnp.int32),
)

_NROWS = 1000 * 1000
_WBLK = 8192


def _wt_body(wt_ref, o_ref):
    o_ref[...] = wt_ref[...].T


_wtrans = pl.pallas_call(
    _wt_body,
    grid=(pl.cdiv(_NROWS, _WBLK),),
    in_specs=[pl.BlockSpec((_EMBED, _WBLK), lambda i: (0, i))],
    out_specs=pl.BlockSpec((_WBLK, _EMBED), lambda i: (i, 0)),
    out_shape=jax.ShapeDtypeStruct((_NROWS, _EMBED), jnp.float32),
)


@functools.partial(
    pl.kernel,
    mesh=_mesh,
    out_type=jax.ShapeDtypeStruct((_TOTAL * _EMBED,), jnp.float32),
    scratch_types=[
        pltpu.VMEM((_PER_W, 128), jnp.int32),
        pltpu.VMEM((_GDEPTH * 128, _EMBED), jnp.float32),
        pltpu.VMEM((_ODEPTH * 4096,), jnp.float32),
        pltpu.SemaphoreType.DMA,
        pltpu.SemaphoreType.DMA,
        pltpu.SemaphoreType.DMA,
    ],
    compiler_params=pltpu.CompilerParams(
        use_tc_tiling_on_sc=False, needs_layout_passes=False
    ),
)
def _lookup(idx_hbm, table_hbm, out_hbm, idxv, rows, rowsT,
            sem_in, sem_g, sem_out):
    wid = lax.axis_index("s") * _NC + lax.axis_index("c")
    t0 = wid * _PER_W
    lanes = lax.iota(jnp.int32, 16)
    nvecs = [n0 + lanes for n0 in range(0, 128, 16)]
    zeros = jnp.zeros((16,), jnp.int32)

    pltpu.async_copy(idx_hbm.at[pl.ds(t0, _PER_W), :], idxv, sem_in).wait()

    def fire_gather(t, slot):
        return pltpu.async_copy(
            table_hbm.at[idxv.at[t]],
            rows.at[pl.ds(slot * 128, 128), :], sem_g)

    def transpose(slot, oslot):
        rvecs = [slot * 128 + nv for nv in nvecs]
        ovecs = [oslot * 4096 + nv for nv in nvecs]

        def tr_body(c, carry):
            dv = (lanes + c) & 31
            pv = dv * 128
            vs = [plsc.load_gather(rows, [rvecs[i], dv]) for i in range(8)]
            for i in range(8):
                plsc.store_scatter(rowsT, [pv + ovecs[i]], vs[i])
            return carry

        lax.fori_loop(0, _EMBED, tr_body, 0, unroll=2)

    def fire_outs(t, oslot):
        copies = []
        p = t >> 7
        nb = t & 127
        r0 = p * (4 * _NB) + nb
        for db in range(4):
            copies.append(pltpu.async_copy(
                rowsT.at[pl.ds(oslot * 4096 + db * 1024, 1024)],
                out_hbm.at[pl.ds((r0 + db * _NB) * 1024, 1024)],
                sem_out))
        return copies

    prime = [fire_gather(t, t) for t in range(_GDEPTH)]
    for t in range(_ODEPTH):
        prime[t].wait()
        transpose(t, t)
        fire_outs(t0 + t, t)
        fire_gather(t + _GDEPTH, t)

    def steady(t, carry):
        slot = t & (_GDEPTH - 1)
        oslot = t & (_ODEPTH - 1)
        pltpu.make_async_copy(
            table_hbm.at[idxv.at[t]],
            rows.at[pl.ds(slot * 128, 128), :], sem_g).wait()
        pltpu.make_async_copy(
            rowsT.at[pl.ds(0, 4096)], out_hbm.at[pl.ds(0, 4096)], sem_out).wait()
        transpose(slot, oslot)
        fire_outs(t0 + t, oslot)
        fire_gather(t + _GDEPTH, slot)
        return carry

    lax.fori_loop(_ODEPTH, _PER_W - _GDEPTH, steady, 0)

    for t in range(_PER_W - _GDEPTH, _PER_W):
        slot = t % _GDEPTH
        oslot = t % _ODEPTH
        pltpu.make_async_copy(
            table_hbm.at[idxv.at[t]],
            rows.at[pl.ds(slot * 128, 128), :], sem_g).wait()
        pltpu.make_async_copy(
            rowsT.at[pl.ds(0, 4096)], out_hbm.at[pl.ds(0, 4096)], sem_out).wait()
        transpose(slot, oslot)
        fire_outs(t0 + t, oslot)

    for _ in range(_ODEPTH):
        pltpu.make_async_copy(
            rowsT.at[pl.ds(0, 4096)], out_hbm.at[pl.ds(0, 4096)], sem_out).wait()


def kernel(coords, W):
    c3 = coords.transpose(1, 2, 0).reshape(_P, 2, _NB, 128)
    c3 = c3.transpose(0, 2, 1, 3).reshape(_TCOLS, 2, 128)
    idx = _idx_tc(c3)
    w_rm = _wtrans(W.T)
    out = _lookup(idx, w_rm)
    out = out.reshape(_P, 4, _NB, 8, 128)
    out = out.transpose(2, 4, 0, 1, 3).reshape(_B, _P, _EMBED)
    return out


# speedup vs baseline: 1.2485x; 1.1567x over previous
import functools
import math

import jax
import jax.numpy as jnp
from jax import lax
from jax.experimental import pallas as pl
from jax.experimental.pallas import tpu as pltpu
from jax.experimental.pallas import tpu_sc as plsc

_INTERVAL = 0.001
_NUM_COL = int(math.ceil(1.0 / _INTERVAL))
_EMBED = 32
_B = 16384
_P = 50
_TOTAL = _B * _P

_NC = 2
_NS = 16
_NW = _NC * _NS

_NB = _B // 128
_TCOLS = _P * _NB
_PER_W = _TCOLS // _NW

_GDEPTH = 16
_ODEPTH = 8

_mesh = plsc.VectorSubcoreMesh(core_axis_name="c", subcore_axis_name="s")


def _idx_body(c_ref, o_ref):
    t = jnp.floor(c_ref[...] / _INTERVAL)
    t = jnp.clip(t, 0, _NUM_COL - 1).astype(jnp.int32)
    o_ref[...] = t[:, 1, :] * _NUM_COL + t[:, 0, :]


_idx_tc = pl.pallas_call(
    _idx_body,
    grid=(8,),
    in_specs=[
        pl.BlockSpec((_TCOLS // 8, 2, 128), lambda i: (i, 0, 0)),
    ],
    out_specs=pl.BlockSpec((_TCOLS // 8, 128), lambda i: (i, 0)),
    out_shape=jax.ShapeDtypeStruct((_TCOLS, 128), jnp.int32),
)

_NROWS = 1000 * 1000
_WBLK = 8192


def _wt_body(wt_ref, o_ref):
    o_ref[...] = wt_ref[...].T.reshape(_WBLK // 4, 128)


_wtrans = pl.pallas_call(
    _wt_body,
    grid=(pl.cdiv(_NROWS, _WBLK),),
    in_specs=[pl.BlockSpec((_EMBED, _WBLK), lambda i: (0, i))],
    out_specs=pl.BlockSpec((_WBLK // 4, 128), lambda i: (i, 0)),
    out_shape=jax.ShapeDtypeStruct((_NROWS // 4, 128), jnp.float32),
)


@functools.partial(
    pl.kernel,
    mesh=_mesh,
    out_type=jax.ShapeDtypeStruct((_TOTAL * _EMBED,), jnp.float32),
    scratch_types=[
        pltpu.VMEM((_PER_W, 128), jnp.int32),
        pltpu.VMEM((_GDEPTH * 128, _EMBED), jnp.float32),
        pltpu.VMEM((_ODEPTH * 4096,), jnp.float32),
        pltpu.SemaphoreType.DMA,
        pltpu.SemaphoreType.DMA,
        pltpu.SemaphoreType.DMA,
    ],
    compiler_params=pltpu.CompilerParams(
        use_tc_tiling_on_sc=False, needs_layout_passes=False
    ),
)
def _lookup(idx_hbm, table_hbm, out_hbm, idxv, rows, rowsT,
            sem_in, sem_g, sem_out):
    wid = lax.axis_index("s") * _NC + lax.axis_index("c")
    t0 = wid * _PER_W
    lanes = lax.iota(jnp.int32, 16)
    nvecs = [n0 + lanes for n0 in range(0, 128, 16)]
    zeros = jnp.zeros((16,), jnp.int32)

    pltpu.async_copy(idx_hbm.at[pl.ds(t0, _PER_W), :], idxv, sem_in).wait()

    def fire_gather(t, slot):
        return pltpu.async_copy(
            table_hbm.at[idxv.at[t]],
            rows.at[pl.ds(slot * 128, 128), :], sem_g)

    def transpose(slot, oslot):
        rvecs = [slot * 128 + nv for nv in nvecs]
        ovecs = [oslot * 4096 + nv for nv in nvecs]

        def tr_body(c, carry):
            dv = (lanes + c) & 31
            pv = dv * 128
            vs = [plsc.load_gather(rows, [rvecs[i], dv]) for i in range(8)]
            for i in range(8):
                plsc.store_scatter(rowsT, [pv + ovecs[i]], vs[i])
            return carry

        lax.fori_loop(0, _EMBED, tr_body, 0, unroll=2)

    def fire_outs(t, oslot):
        copies = []
        p = t >> 7
        nb = t & 127
        r0 = p * (4 * _NB) + nb
        for db in range(4):
            copies.append(pltpu.async_copy(
                rowsT.at[pl.ds(oslot * 4096 + db * 1024, 1024)],
                out_hbm.at[pl.ds((r0 + db * _NB) * 1024, 1024)],
                sem_out))
        return copies

    prime = [fire_gather(t, t) for t in range(_GDEPTH)]
    for t in range(_ODEPTH):
        prime[t].wait()
        transpose(t, t)
        fire_outs(t0 + t, t)
        fire_gather(t + _GDEPTH, t)

    def steady(t, carry):
        slot = t & (_GDEPTH - 1)
        oslot = t & (_ODEPTH - 1)
        pltpu.make_async_copy(
            table_hbm.at[idxv.at[t]],
            rows.at[pl.ds(slot * 128, 128), :], sem_g).wait()
        pltpu.make_async_copy(
            rowsT.at[pl.ds(0, 4096)], out_hbm.at[pl.ds(0, 4096)], sem_out).wait()
        transpose(slot, oslot)
        fire_outs(t0 + t, oslot)
        fire_gather(t + _GDEPTH, slot)
        return carry

    lax.fori_loop(_ODEPTH, _PER_W - _GDEPTH, steady, 0)

    for t in range(_PER_W - _GDEPTH, _PER_W):
        slot = t % _GDEPTH
        oslot = t % _ODEPTH
        pltpu.make_async_copy(
            table_hbm.at[idxv.at[t]],
            rows.at[pl.ds(slot * 128, 128), :], sem_g).wait()
        pltpu.make_async_copy(
            rowsT.at[pl.ds(0, 4096)], out_hbm.at[pl.ds(0, 4096)], sem_out).wait()
        transpose(slot, oslot)
        fire_outs(t0 + t, oslot)

    for _ in range(_ODEPTH):
        pltpu.make_async_copy(
            rowsT.at[pl.ds(0, 4096)], out_hbm.at[pl.ds(0, 4096)], sem_out).wait()


def kernel(coords, W):
    c3 = coords.transpose(1, 2, 0).reshape(_P, 2, _NB, 128)
    c3 = c3.transpose(0, 2, 1, 3).reshape(_TCOLS, 2, 128)
    idx = _idx_tc(c3)
    out = _lookup(idx, W)
    out = out.reshape(_P, 4, _NB, 8, 128)
    out = out.transpose(2, 4, 0, 1, 3).reshape(_B, _P, _EMBED)
    return out


# speedup vs baseline: 1.2485x; 1.0000x over previous
import functools
import math

import jax
import jax.numpy as jnp
from jax import lax
from jax.experimental import pallas as pl
from jax.experimental.pallas import tpu as pltpu
from jax.experimental.pallas import tpu_sc as plsc

_INTERVAL = 0.001
_NUM_COL = int(math.ceil(1.0 / _INTERVAL))
_EMBED = 32
_B = 16384
_P = 50
_TOTAL = _B * _P

_NC = 2
_NS = 16
_NW = _NC * _NS

_NB = _B // 128
_TCOLS = _P * _NB
_PER_W = _TCOLS // _NW

_GDEPTH = 16
_ODEPTH = 8

_mesh = plsc.VectorSubcoreMesh(core_axis_name="c", subcore_axis_name="s")


def _idx_body(c_ref, o_ref):
    t = jnp.floor(c_ref[...] / _INTERVAL)
    t = jnp.clip(t, 0, _NUM_COL - 1).astype(jnp.int32)
    o_ref[...] = t[:, 1, :] * _NUM_COL + t[:, 0, :]


_idx_tc = pl.pallas_call(
    _idx_body,
    grid=(8,),
    in_specs=[
        pl.BlockSpec((_TCOLS // 8, 2, 128), lambda i: (i, 0, 0)),
    ],
    out_specs=pl.BlockSpec((_TCOLS // 8, 128), lambda i: (i, 0)),
    out_shape=jax.ShapeDtypeStruct((_TCOLS, 128), jnp.int32),
)

@functools.partial(
    pl.kernel,
    mesh=_mesh,
    out_type=jax.ShapeDtypeStruct((_TOTAL * _EMBED,), jnp.float32),
    scratch_types=[
        pltpu.VMEM((_PER_W, 128), jnp.int32),
        pltpu.VMEM((_GDEPTH * 128, _EMBED), jnp.float32),
        pltpu.VMEM((_ODEPTH * 4096,), jnp.float32),
        pltpu.SemaphoreType.DMA,
        pltpu.SemaphoreType.DMA,
        pltpu.SemaphoreType.DMA,
    ],
    compiler_params=pltpu.CompilerParams(
        use_tc_tiling_on_sc=False, needs_layout_passes=False
    ),
)
def _lookup(idx_hbm, table_hbm, out_hbm, idxv, rows, rowsT,
            sem_in, sem_g, sem_out):
    wid = lax.axis_index("s") * _NC + lax.axis_index("c")
    t0 = wid * _PER_W
    lanes = lax.iota(jnp.int32, 16)
    nvecs = [n0 + lanes for n0 in range(0, 128, 16)]

    pltpu.async_copy(idx_hbm.at[pl.ds(t0, _PER_W), :], idxv, sem_in).wait()

    def fire_gather(t, slot):
        return pltpu.async_copy(
            table_hbm.at[idxv.at[t]],
            rows.at[pl.ds(slot * 128, 128), :], sem_g)

    def transpose(slot, oslot):
        rvecs = [slot * 128 + nv for nv in nvecs]
        ovecs = [oslot * 4096 + nv for nv in nvecs]

        def tr_body(c, carry):
            dv = (lanes + c) & 31
            pv = dv * 128
            vs = [plsc.load_gather(rows, [rvecs[i], dv]) for i in range(8)]
            for i in range(8):
                plsc.store_scatter(rowsT, [pv + ovecs[i]], vs[i])
            return carry

        lax.fori_loop(0, _EMBED, tr_body, 0, unroll=2)

    def fire_outs(t, oslot):
        copies = []
        p = t >> 7
        nb = t & 127
        r0 = p * (4 * _NB) + nb
        for db in range(4):
            copies.append(pltpu.async_copy(
                rowsT.at[pl.ds(oslot * 4096 + db * 1024, 1024)],
                out_hbm.at[pl.ds((r0 + db * _NB) * 1024, 1024)],
                sem_out))
        return copies

    prime = [fire_gather(t, t) for t in range(_GDEPTH)]
    for t in range(_ODEPTH):
        prime[t].wait()
        transpose(t, t)
        fire_outs(t0 + t, t)
        fire_gather(t + _GDEPTH, t)

    def steady(t, carry):
        slot = t & (_GDEPTH - 1)
        oslot = t & (_ODEPTH - 1)
        pltpu.make_async_copy(
            table_hbm.at[idxv.at[t]],
            rows.at[pl.ds(slot * 128, 128), :], sem_g).wait()
        pltpu.make_async_copy(
            rowsT.at[pl.ds(0, 4096)], out_hbm.at[pl.ds(0, 4096)], sem_out).wait()
        transpose(slot, oslot)
        fire_outs(t0 + t, oslot)
        fire_gather(t + _GDEPTH, slot)
        return carry

    lax.fori_loop(_ODEPTH, _PER_W - _GDEPTH, steady, 0)

    for t in range(_PER_W - _GDEPTH, _PER_W):
        slot = t % _GDEPTH
        oslot = t % _ODEPTH
        pltpu.make_async_copy(
            table_hbm.at[idxv.at[t]],
            rows.at[pl.ds(slot * 128, 128), :], sem_g).wait()
        pltpu.make_async_copy(
            rowsT.at[pl.ds(0, 4096)], out_hbm.at[pl.ds(0, 4096)], sem_out).wait()
        transpose(slot, oslot)
        fire_outs(t0 + t, oslot)

    for _ in range(_ODEPTH):
        pltpu.make_async_copy(
            rowsT.at[pl.ds(0, 4096)], out_hbm.at[pl.ds(0, 4096)], sem_out).wait()


def kernel(coords, W):
    c3 = coords.transpose(1, 2, 0).reshape(_P, 2, _NB, 128)
    c3 = c3.transpose(0, 2, 1, 3).reshape(_TCOLS, 2, 128)
    idx = _idx_tc(c3)
    out = _lookup(idx, W)
    out = out.reshape(_P, 4, _NB, 8, 128)
    out = out.transpose(2, 4, 0, 1, 3).reshape(_B, _P, _EMBED)
    return out
